# Initial kernel scaffold; baseline (speedup 1.0000x reference)
import jax
import jax.numpy as jnp
from jax.experimental import pallas as pl


def kernel(node_feats, segment_ids, sequence):
    raise NotImplementedError("write your pallas kernel here")


# speedup vs baseline: 5.6152x; 5.6152x over previous
import functools

import jax
import jax.numpy as jnp
from jax import lax
from jax.experimental import pallas as pl
from jax.experimental.pallas import tpu as pltpu
from jax.experimental.pallas import tpu_sc as plsc

N_NODES = 800000
N_GRAPHS = 100000
D = 32
B = 4096
L = 200

NC = 2
NS = 16
LANES = 16

QS = 25000
QCAP = 25008
DUMP = QCAP
QROWS = QCAP + 8
TBL = 100008
CH = 512
CW = 128
NFC = 196
SEQ = B * L
ROWS_W = SEQ // (NC * NS)
GCH = 512
IOTA16 = None


def _iota16():
  return lax.iota(jnp.int32, 16)


def _p1_body(feats, seg, nb, table, sp_feat, sp_cnt, bvm, segb, featb,
             idx2, onesb, fbuf, cbuf, ibuf, zf, zc):
  c = lax.axis_index("c")
  w = lax.axis_index("s")
  it = _iota16()

  pltpu.sync_copy(nb, bvm)
  bv = bvm[pl.ds(0, 16)]

  def _fill(r, _):
    onesb[r, pl.ds(0, 16)] = jnp.where(it == 0, 1.0, 0.0)
    return 0
  lax.fori_loop(0, CH, _fill, 0)

  def _zfill(r, _):
    zf[r, pl.ds(0, 16)] = jnp.zeros((16,), jnp.float32)
    zf[r, pl.ds(16, 16)] = jnp.zeros((16,), jnp.float32)
    zc[r, pl.ds(0, 16)] = jnp.zeros((16,), jnp.float32)
    return 0
  lax.fori_loop(0, CW, _zfill, 0)

  flo = (w * NFC) // NS
  fhi = ((w + 1) * NFC) // NS

  for p_local in range(2):
    p = c * 2 + p_local
    qlo = p * QS
    qn = jnp.where(p == 3, QCAP, QS)
    nlo = jnp.where(c == 0, bv[p_local], bv[2 + p_local])
    nhi = jnp.where(c == 0, bv[p_local + 1], bv[3 + p_local])

    def _zero(i, _):
      st = jnp.minimum(i * CW, QROWS - CW)
      pltpu.sync_copy(zf, sp_feat.at[pl.ds(st, CW)])
      pltpu.sync_copy(zc, sp_cnt.at[pl.ds(st, CW)])
      return 0
    lax.fori_loop(flo, fhi, _zero, 0)
    plsc.subcore_barrier()

    lo_w = nlo + (nhi - nlo) * w // NS
    hi_w = nlo + (nhi - nlo) * (w + 1) // NS
    a_lo = (lo_w // 8) * 8
    nch = jnp.maximum(0, (hi_w - a_lo + CH - 1) // CH)

    def _chunk(i, _):
      raw = a_lo + i * CH
      st = jnp.minimum(raw, N_NODES - CH)
      pltpu.sync_copy(seg.at[pl.ds(st, CH)], segb)
      pltpu.sync_copy(feats.at[pl.ds(st, CH)], featb)
      for k in range(CH // 16):
        v = segb[pl.ds(k * 16, 16)]
        pos = st + k * 16 + it
        valid = (pos >= lo_w) & (pos >= raw) & (pos < hi_w)
        idx = jnp.where(valid, v - qlo, DUMP)
        idx2[k // 8, pl.ds((k % 8) * 16, 16)] = idx
      for j in range(CH // CW):
        pltpu.sync_copy(featb.at[pl.ds(j * CW, CW)],
                        sp_feat.at[idx2.at[j]], add=True)
        pltpu.sync_copy(onesb.at[pl.ds(j * CW, CW)],
                        sp_cnt.at[idx2.at[j]], add=True)
      return 0
    lax.fori_loop(0, nch, _chunk, 0)
    plsc.subcore_barrier()

    def _fin(i, _):
      st = jnp.minimum(i * CW, qn - CW)
      pltpu.sync_copy(sp_feat.at[pl.ds(st, CW)], fbuf)
      pltpu.sync_copy(sp_cnt.at[pl.ds(st, CW)], cbuf)
      for g in range(CW):
        crow = cbuf[g, pl.ds(0, 16)]
        inv = 1.0 / jnp.maximum(crow, 1.0)
        sp = jnp.full((16,), inv[0])
        fbuf[g, pl.ds(0, 16)] = fbuf[g, pl.ds(0, 16)] * sp
        fbuf[g, pl.ds(16, 16)] = fbuf[g, pl.ds(16, 16)] * sp
      pltpu.sync_copy(fbuf, table.at[pl.ds(qlo + st, CW)])
      return 0
    lax.fori_loop(flo, fhi, _fin, 0)
    plsc.subcore_barrier()


def _p2_body(table, seqi, out, sidx, idx2, rows, sem):
  c = lax.axis_index("c")
  w = lax.axis_index("s")
  wid = w * NC + c
  base_w = wid * ROWS_W

  def _chunk(i, _):
    base = base_w + i * GCH
    pltpu.sync_copy(seqi.at[pl.ds(base, GCH)], sidx)
    for k in range(GCH // 16):
      v = sidx[pl.ds(k * 16, 16)] - 1
      v = jnp.where(v < 0, N_GRAPHS, v)
      idx2[k // 8, pl.ds((k % 8) * 16, 16)] = v
    descs = []
    for j in range(GCH // CW):
      descs.append(pltpu.async_copy(table.at[idx2.at[j]],
                                    rows.at[pl.ds(j * CW, CW)], sem))
    for d in descs:
      d.wait()
    pltpu.sync_copy(rows, out.at[pl.ds(base, GCH)])
    return 0
  lax.fori_loop(0, ROWS_W // GCH, _chunk, 0)


def kernel(node_feats, segment_ids, sequence):
  seg = segment_ids.astype(jnp.int32)
  seqf = sequence.astype(jnp.int32).reshape(-1)
  nb = jnp.searchsorted(seg, jnp.array([QS, 2 * QS, 3 * QS], jnp.int32))
  nbv = jnp.zeros((16,), jnp.int32)
  nbv = nbv.at[1:4].set(nb.astype(jnp.int32)).at[4].set(N_NODES)

  mesh = plsc.VectorSubcoreMesh(core_axis_name="c", subcore_axis_name="s",
                                num_cores=NC, num_subcores=NS)
  cparams = pltpu.CompilerParams(use_tc_tiling_on_sc=False)

  p1 = pl.kernel(
      _p1_body,
      out_type=jax.ShapeDtypeStruct((TBL, D), jnp.float32),
      mesh=mesh,
      compiler_params=cparams,
      scratch_types=[
          pltpu.VMEM_SHARED((QROWS, D), jnp.float32),
          pltpu.VMEM_SHARED((QROWS, 16), jnp.float32),
          pltpu.VMEM((16,), jnp.int32),
          pltpu.VMEM((CH,), jnp.int32),
          pltpu.VMEM((CH, D), jnp.float32),
          pltpu.VMEM((CH // CW, CW), jnp.int32),
          pltpu.VMEM((CH, 16), jnp.float32),
          pltpu.VMEM((CW, D), jnp.float32),
          pltpu.VMEM((CW, 16), jnp.float32),
          pltpu.VMEM((16,), jnp.float32),
          pltpu.VMEM((CW, D), jnp.float32),
          pltpu.VMEM((CW, 16), jnp.float32),
      ],
  )
  table = p1(node_feats, seg, nbv)

  p2 = pl.kernel(
      _p2_body,
      out_type=jax.ShapeDtypeStruct((SEQ, D), jnp.float32),
      mesh=mesh,
      compiler_params=cparams,
      scratch_types=[
          pltpu.VMEM((GCH,), jnp.int32),
          pltpu.VMEM((GCH // CW, CW), jnp.int32),
          pltpu.VMEM((GCH, D), jnp.float32),
          pltpu.SemaphoreType.DMA,
      ],
  )
  outf = p2(table, seqf)
  return outf.reshape(B, L, D)


# speedup vs baseline: 6.4186x; 1.1431x over previous
import jax
import jax.numpy as jnp
from jax import lax
from jax.experimental import pallas as pl
from jax.experimental.pallas import tpu as pltpu
from jax.experimental.pallas import tpu_sc as plsc

N_NODES = 800000
N_GRAPHS = 100000
D = 32
B = 4096
L = 200

NC = 2
NS = 16

QS = 25000
QCAP = 25008
DUMP = QCAP
QROWS = QCAP + 8
TBL = 100008
CH = 256
CW = 128
NFC = 196
SEQ = B * L
ROWS_W = SEQ // (NC * NS)
GCH = 512
NGC = ROWS_W // GCH


def _iota16():
  return lax.iota(jnp.int32, 16)


def _transform_idx(segb, idx2, st, raw, lo_w, hi_w, qlo, n, it):
  for k in range(n // 16):
    v = segb[pl.ds(k * 16, 16)]
    pos = st + k * 16 + it
    valid = (pos >= lo_w) & (pos >= raw) & (pos < hi_w)
    idx = jnp.where(valid, v - qlo, DUMP)
    idx2[k // 8, pl.ds((k % 8) * 16, 16)] = idx


def _p1_body(feats, seg, nb, table, sp_feat, sp_cnt, bvm,
             segb0, segb1, featb0, featb1, idx20, idx21, onesb,
             fbuf, cbuf, lsem0, lsem1, asem0, asem1, zsem):
  c = lax.axis_index("c")
  w = lax.axis_index("s")
  it = _iota16()
  segbs, featbs, idx2s = (segb0, segb1), (featb0, featb1), (idx20, idx21)
  lsems, asems = (lsem0, lsem1), (asem0, asem1)

  pltpu.sync_copy(nb, bvm)
  bv = bvm[pl.ds(0, 16)]

  def _fill(r, _):
    onesb[r, pl.ds(0, 16)] = jnp.where(it == 0, 1.0, 0.0)
    cbuf[r, pl.ds(0, 16)] = jnp.zeros((16,), jnp.float32)
    return 0
  lax.fori_loop(0, CH, _fill, 0)

  def _zfill(r, _):
    fbuf[r, pl.ds(0, 16)] = jnp.zeros((16,), jnp.float32)
    fbuf[r, pl.ds(16, 16)] = jnp.zeros((16,), jnp.float32)
    return 0
  lax.fori_loop(0, CW, _zfill, 0)

  flo = (w * NFC) // NS
  fhi = ((w + 1) * NFC) // NS

  for p_local in range(2):
    p = c * 2 + p_local
    qlo = p * QS
    qn = jnp.where(p == 3, QCAP, QS)
    nlo = jnp.where(c == 0, bv[p_local], bv[2 + p_local])
    nhi = jnp.where(c == 0, bv[p_local + 1], bv[3 + p_local])

    def _zero(i, _):
      st = jnp.minimum(i * CW, QROWS - CW)
      pltpu.async_copy(fbuf, sp_feat.at[pl.ds(st, CW)], zsem)
      pltpu.async_copy(cbuf, sp_cnt.at[pl.ds(st, CW)], zsem)
      return 0
    lax.fori_loop(flo, fhi, _zero, 0)

    def _zdrain(i, _):
      pltpu.make_async_copy(fbuf, sp_feat.at[pl.ds(0, CW)], zsem).wait()
      pltpu.make_async_copy(cbuf, sp_cnt.at[pl.ds(0, CW)], zsem).wait()
      return 0
    lax.fori_loop(flo, fhi, _zdrain, 0)
    plsc.subcore_barrier()

    lo_w = nlo + (nhi - nlo) * w // NS
    hi_w = nlo + (nhi - nlo) * (w + 1) // NS
    a_lo = (lo_w // 8) * 8
    nch = jnp.maximum(0, (hi_w - a_lo + CH - 1) // CH)

    def _fire_load(ci, par):
      st = jnp.minimum(a_lo + ci * CH, N_NODES - CH)
      pltpu.async_copy(seg.at[pl.ds(st, CH)], segbs[par], lsems[par])
      pltpu.async_copy(feats.at[pl.ds(st, CH)], featbs[par], lsems[par])

    def _wait_load(par):
      pltpu.make_async_copy(seg.at[pl.ds(0, CH)], segbs[par],
                            lsems[par]).wait()
      pltpu.make_async_copy(feats.at[pl.ds(0, CH)], featbs[par],
                            lsems[par]).wait()

    def _drain_adds(par):
      for _ in range(CH // CW):
        pltpu.make_async_copy(feats.at[pl.ds(0, CW)],
                              featbs[par].at[pl.ds(0, CW)],
                              asems[par]).wait()
        pltpu.make_async_copy(feats.at[pl.ds(0, CW)],
                              onesb.at[pl.ds(0, CW)], asems[par]).wait()

    @pl.when(nch > 0)
    def _():
      _fire_load(0, 0)
    @pl.when(nch > 1)
    def _():
      _fire_load(1, 1)

    def _chunk2(i2, _):
      for par in range(2):
        ci = i2 * 2 + par

        @pl.when(ci < nch)
        def _():
          raw = a_lo + ci * CH
          st = jnp.minimum(raw, N_NODES - CH)
          _wait_load(par)
          _transform_idx(segbs[par], idx2s[par], st, raw, lo_w, hi_w,
                         qlo, CH, it)
          for j in range(CH // CW):
            pltpu.async_copy(featbs[par].at[pl.ds(j * CW, CW)],
                             sp_feat.at[idx2s[par].at[j]],
                             asems[par], add=True)
            pltpu.async_copy(onesb.at[pl.ds(j * CW, CW)],
                             sp_cnt.at[idx2s[par].at[j]],
                             asems[par], add=True)

        @pl.when(ci + 2 < nch)
        def _():
          _drain_adds(par)
          _fire_load(ci + 2, par)
      return 0
    lax.fori_loop(0, (nch + 1) // 2, _chunk2, 0)

    for par in range(2):
      @pl.when(nch >= (2 if par else 1))
      def _(par=par):
        _drain_adds(par)
    plsc.subcore_barrier()

    def _fin(i, _):
      st = jnp.minimum(i * CW, qn - CW)
      pltpu.sync_copy(sp_feat.at[pl.ds(st, CW)], fbuf)
      pltpu.sync_copy(sp_cnt.at[pl.ds(st, CW)], cbuf)
      for g in range(CW):
        crow = cbuf[g, pl.ds(0, 16)]
        inv = 1.0 / jnp.maximum(crow, 1.0)
        sp = jnp.full((16,), inv[0])
        fbuf[g, pl.ds(0, 16)] = fbuf[g, pl.ds(0, 16)] * sp
        fbuf[g, pl.ds(16, 16)] = fbuf[g, pl.ds(16, 16)] * sp
      pltpu.sync_copy(fbuf, table.at[pl.ds(qlo + st, CW)])
      return 0
    lax.fori_loop(flo, fhi, _fin, 0)

    @pl.when(p_local == 0)
    def _():
      def _refill(r, _):
        fbuf[r, pl.ds(0, 16)] = jnp.zeros((16,), jnp.float32)
        fbuf[r, pl.ds(16, 16)] = jnp.zeros((16,), jnp.float32)
        cbuf[r, pl.ds(0, 16)] = jnp.zeros((16,), jnp.float32)
        return 0
      lax.fori_loop(0, CW, _refill, 0)
    plsc.subcore_barrier()


def _p2_body(table, seqi, out, sidx0, sidx1, idx20, idx21, rows0, rows1,
             lsem0, lsem1, gsem0, gsem1, osem0, osem1):
  c = lax.axis_index("c")
  w = lax.axis_index("s")
  wid = w * NC + c
  base_w = wid * ROWS_W
  sidxs, idx2s, rowss = (sidx0, sidx1), (idx20, idx21), (rows0, rows1)
  lsems, gsems, osems = (lsem0, lsem1), (gsem0, gsem1), (osem0, osem1)

  def _fire_load(ci, par):
    pltpu.async_copy(seqi.at[pl.ds(base_w + ci * GCH, GCH)], sidxs[par],
                     lsems[par])

  _fire_load(0, 0)
  _fire_load(1, 1)

  def _chunk2(i2, _):
    for par in range(2):
      ci = i2 * 2 + par
      pltpu.make_async_copy(seqi.at[pl.ds(0, GCH)], sidxs[par],
                            lsems[par]).wait()
      for k in range(GCH // 16):
        v = sidxs[par][pl.ds(k * 16, 16)] - 1
        v = jnp.where(v < 0, N_GRAPHS, v)
        idx2s[par][k // 8, pl.ds((k % 8) * 16, 16)] = v

      @pl.when(ci + 2 < NGC)
      def _():
        _fire_load(ci + 2, par)

      @pl.when(ci >= 2)
      def _():
        pltpu.make_async_copy(rowss[par], out.at[pl.ds(0, GCH)],
                              osems[par]).wait()
      for j in range(GCH // CW):
        pltpu.async_copy(table.at[idx2s[par].at[j]],
                         rowss[par].at[pl.ds(j * CW, CW)], gsems[par])
      for j in range(GCH // CW):
        pltpu.make_async_copy(table.at[pl.ds(0, CW)],
                              rowss[par].at[pl.ds(0, CW)],
                              gsems[par]).wait()
      pltpu.async_copy(rowss[par], out.at[pl.ds(base_w + ci * GCH, GCH)],
                       osems[par])
    return 0
  lax.fori_loop(0, NGC // 2, _chunk2, 0)
  for par in range(2):
    pltpu.make_async_copy(rowss[par], out.at[pl.ds(0, GCH)],
                          osems[par]).wait()


def kernel(node_feats, segment_ids, sequence):
  seg = segment_ids.astype(jnp.int32)
  seqf = sequence.astype(jnp.int32).reshape(-1)
  nb = jnp.searchsorted(seg, jnp.array([QS, 2 * QS, 3 * QS], jnp.int32))
  nbv = jnp.zeros((16,), jnp.int32)
  nbv = nbv.at[1:4].set(nb.astype(jnp.int32)).at[4].set(N_NODES)

  mesh = plsc.VectorSubcoreMesh(core_axis_name="c", subcore_axis_name="s",
                                num_cores=NC, num_subcores=NS)
  cparams = pltpu.CompilerParams(use_tc_tiling_on_sc=False)

  p1 = pl.kernel(
      _p1_body,
      out_type=jax.ShapeDtypeStruct((TBL, D), jnp.float32),
      mesh=mesh,
      compiler_params=cparams,
      scratch_types=[
          pltpu.VMEM_SHARED((QROWS, D), jnp.float32),
          pltpu.VMEM_SHARED((QROWS, 16), jnp.float32),
          pltpu.VMEM((16,), jnp.int32),
          pltpu.VMEM((CH,), jnp.int32),
          pltpu.VMEM((CH,), jnp.int32),
          pltpu.VMEM((CH, D), jnp.float32),
          pltpu.VMEM((CH, D), jnp.float32),
          pltpu.VMEM((CH // CW, CW), jnp.int32),
          pltpu.VMEM((CH // CW, CW), jnp.int32),
          pltpu.VMEM((CH, 16), jnp.float32),
          pltpu.VMEM((CW, D), jnp.float32),
          pltpu.VMEM((CW, 16), jnp.float32),
          pltpu.SemaphoreType.DMA,
          pltpu.SemaphoreType.DMA,
          pltpu.SemaphoreType.DMA,
          pltpu.SemaphoreType.DMA,
          pltpu.SemaphoreType.DMA,
      ],
  )
  table = p1(node_feats, seg, nbv)

  p2 = pl.kernel(
      _p2_body,
      out_type=jax.ShapeDtypeStruct((SEQ, D), jnp.float32),
      mesh=mesh,
      compiler_params=cparams,
      scratch_types=[
          pltpu.VMEM((GCH,), jnp.int32),
          pltpu.VMEM((GCH,), jnp.int32),
          pltpu.VMEM((GCH // CW, CW), jnp.int32),
          pltpu.VMEM((GCH // CW, CW), jnp.int32),
          pltpu.VMEM((GCH, D), jnp.float32),
          pltpu.VMEM((GCH, D), jnp.float32),
          pltpu.SemaphoreType.DMA,
          pltpu.SemaphoreType.DMA,
          pltpu.SemaphoreType.DMA,
          pltpu.SemaphoreType.DMA,
          pltpu.SemaphoreType.DMA,
          pltpu.SemaphoreType.DMA,
      ],
  )
  outf = p2(table, seqf)
  return outf.reshape(B, L, D)
